# Initial kernel scaffold; baseline (speedup 1.0000x reference)
import jax
import jax.numpy as jnp
from jax.experimental import pallas as pl


def kernel(h_table, r_table, W1, b1, W2, b2, anchors, rel_0, p1_target, p1_rel):
    raise NotImplementedError("write your pallas kernel here")


# speedup vs baseline: 1.1006x; 1.1006x over previous
import functools

import jax
import jax.numpy as jnp
from jax import lax
from jax.experimental import pallas as pl
from jax.experimental.pallas import tpu as pltpu
from jax.experimental.pallas import tpu_sc as plsc

N_ENT = 1000000
N_REL = 500
DIM = 64
B = 4096
K = 32

NC, NS = 2, 16
NW = NC * NS
CH = 128
NBUF = 8
ROWS_H = B * K
PER_W = ROWS_H // NW
N_ITER = PER_W // (CH * NBUF)
PER_W_B = B // NW

BBLK = 256
GRID = B // BBLK
RPAD = 512


def _sc_gather_body(h_hbm, r_hbm, idx_t, idx_a, idx_r, outh, outa, outr,
                    idxbuf, rb0, rb1, rb2, rb3, rb4, rb5, rb6, rb7,
                    gsem, ssem):
    rowbufs = (rb0, rb1, rb2, rb3, rb4, rb5, rb6, rb7)
    wid = lax.axis_index("s") * NC + lax.axis_index("c")
    base = wid * PER_W

    def loop_body(j, carry):
        off = base + j * (CH * NBUF)
        pltpu.sync_copy(idx_t.at[pl.ds(off, CH * NBUF)], idxbuf)
        handles = [
            pltpu.async_copy(h_hbm.at[idxbuf.at[pl.ds(b * CH, CH)]],
                             rowbufs[b], gsem)
            for b in range(NBUF)
        ]
        for h in handles:
            h.wait()
        stores = [
            pltpu.async_copy(rowbufs[b], outh.at[pl.ds(off + b * CH, CH)],
                             ssem)
            for b in range(NBUF)
        ]
        for s in stores:
            s.wait()
        return carry

    lax.fori_loop(0, N_ITER, loop_body, 0)

    boff = wid * PER_W_B
    pltpu.sync_copy(idx_a.at[pl.ds(boff, PER_W_B)], idxbuf.at[pl.ds(0, PER_W_B)])
    pltpu.sync_copy(idx_r.at[pl.ds(boff, PER_W_B)], idxbuf.at[pl.ds(CH, PER_W_B)])
    ca = pltpu.async_copy(h_hbm.at[idxbuf.at[pl.ds(0, PER_W_B)]], rowbufs[0], gsem)
    cr = pltpu.async_copy(r_hbm.at[idxbuf.at[pl.ds(CH, PER_W_B)]], rowbufs[1], gsem)
    ca.wait()
    cr.wait()
    pltpu.sync_copy(rowbufs[0], outa.at[pl.ds(boff, PER_W_B)])
    pltpu.sync_copy(rowbufs[1], outr.at[pl.ds(boff, PER_W_B)])


@functools.lru_cache(maxsize=None)
def _build_sc_gather():
    return pl.kernel(
        _sc_gather_body,
        out_type=[
            jax.ShapeDtypeStruct((ROWS_H, DIM), jnp.float32),
            jax.ShapeDtypeStruct((B, DIM), jnp.float32),
            jax.ShapeDtypeStruct((B, DIM), jnp.float32),
        ],
        mesh=plsc.VectorSubcoreMesh(core_axis_name="c", subcore_axis_name="s",
                                    num_cores=NC, num_subcores=NS),
        scratch_types=(
            [pltpu.VMEM((CH * NBUF,), jnp.int32)]
            + [pltpu.VMEM((CH, DIM), jnp.float32) for _ in range(NBUF)]
            + [pltpu.SemaphoreType.DMA, pltpu.SemaphoreType.DMA]
        ),
        compiler_params=pltpu.CompilerParams(use_tc_tiling_on_sc=False),
    )


def _tc_body(ht_ref, a_ref, rq_ref, prel_ref, rhi_ref, rlo_ref,
             w1a_ref, w1b_ref, w2_ref, b1_ref, b2_ref, out_ref):
    n = BBLK * K
    prel = prel_ref[0, 0, :]
    iota = lax.broadcasted_iota(jnp.int32, (n, RPAD), 1)
    oh = (iota == prel.reshape(n, 1)).astype(jnp.bfloat16)
    rt = jnp.dot(oh, rhi_ref[...], preferred_element_type=jnp.float32)
    rt = rt + jnp.dot(oh, rlo_ref[...], preferred_element_type=jnp.float32)

    rq = rq_ref[0]
    u0 = lax.dot_general(rq, w1a_ref[...], (((1,), (1,)), ((), ())),
                         preferred_element_type=jnp.float32,
                         precision=lax.Precision.HIGHEST)
    vk = lax.dot_general(rt, w1b_ref[...], (((1,), (1,)), ((), ())),
                         preferred_element_type=jnp.float32,
                         precision=lax.Precision.HIGHEST)
    u0e = jnp.broadcast_to(u0.reshape(BBLK, 1, DIM), (BBLK, K, DIM)).reshape(n, DIM)
    act = jnp.maximum(u0e + vk + b1_ref[...], 0.0)
    m = lax.dot_general(act, w2_ref[...], (((1,), (1,)), ((), ())),
                        preferred_element_type=jnp.float32,
                        precision=lax.Precision.HIGHEST) + b2_ref[...]

    ht = ht_ref[0]
    s1 = jnp.sum((m * ht).reshape(BBLK, K, DIM), axis=1)
    s2 = jnp.sum((m * rt).reshape(BBLK, K, DIM), axis=1)
    a = a_ref[0]
    fre = s1 - a * s2
    query = a * rq
    refn = jnp.sum(jnp.abs(fre), axis=1, keepdims=True)
    qn = jnp.sum(jnp.abs(query), axis=1, keepdims=True)
    out_ref[0] = query + fre / (1e-9 + refn / qn * 2.5)


def _tc_call(ht, a, rq, prel, rhi, rlo, w1a, w1b, w2, b1, b2):
    n = BBLK * K
    return pl.pallas_call(
        _tc_body,
        grid=(GRID,),
        in_specs=[
            pl.BlockSpec((1, n, DIM), lambda i: (i, 0, 0)),
            pl.BlockSpec((1, BBLK, DIM), lambda i: (i, 0, 0)),
            pl.BlockSpec((1, BBLK, DIM), lambda i: (i, 0, 0)),
            pl.BlockSpec((1, 1, n), lambda i: (i, 0, 0)),
            pl.BlockSpec((RPAD, DIM), lambda i: (0, 0)),
            pl.BlockSpec((RPAD, DIM), lambda i: (0, 0)),
            pl.BlockSpec((DIM, DIM), lambda i: (0, 0)),
            pl.BlockSpec((DIM, DIM), lambda i: (0, 0)),
            pl.BlockSpec((DIM, DIM), lambda i: (0, 0)),
            pl.BlockSpec((1, DIM), lambda i: (0, 0)),
            pl.BlockSpec((1, DIM), lambda i: (0, 0)),
        ],
        out_specs=pl.BlockSpec((1, BBLK, DIM), lambda i: (i, 0, 0)),
        out_shape=jax.ShapeDtypeStruct((GRID, BBLK, DIM), jnp.float32),
    )(ht, a, rq, prel, rhi, rlo, w1a, w1b, w2, b1, b2)


def kernel(h_table, r_table, W1, b1, W2, b2, anchors, rel_0, p1_target, p1_rel):
    idx_t = p1_target.reshape(-1).astype(jnp.int32)
    idx_a = anchors.astype(jnp.int32)
    idx_r = rel_0.astype(jnp.int32)
    ht_flat, a_rows, rq_rows = _build_sc_gather()(h_table, r_table, idx_t, idx_a, idx_r)

    rhi = r_table.astype(jnp.bfloat16)
    rlo = (r_table - rhi.astype(jnp.float32)).astype(jnp.bfloat16)
    rhi = jnp.pad(rhi, ((0, RPAD - (N_REL + 1)), (0, 0)))
    rlo = jnp.pad(rlo, ((0, RPAD - (N_REL + 1)), (0, 0)))
    w1a = W1[:, :DIM]
    w1b = W1[:, DIM:]

    out = _tc_call(
        ht_flat.reshape(GRID, BBLK * K, DIM),
        a_rows.reshape(GRID, BBLK, DIM),
        rq_rows.reshape(GRID, BBLK, DIM),
        p1_rel.astype(jnp.int32).reshape(GRID, 1, BBLK * K),
        rhi, rlo, w1a, w1b, W2,
        b1.reshape(1, DIM), b2.reshape(1, DIM),
    )
    return out.reshape(B, DIM)


# speedup vs baseline: 1.2559x; 1.1411x over previous
import functools

import jax
import jax.numpy as jnp
from jax import lax
from jax.experimental import pallas as pl
from jax.experimental.pallas import tpu as pltpu
from jax.experimental.pallas import tpu_sc as plsc

N_ENT = 1000000
N_REL = 500
DIM = 64
B = 4096
K = 32

NC, NS = 2, 16
NW = NC * NS
CH = 128
NBUF = 8
ROWS_H = B * K
PER_W = ROWS_H // NW
N_ITER = PER_W // (CH * NBUF)
PER_W_B = B // NW

BBLK = 256
GRID = B // BBLK
NPAIR = BBLK * K // 2
RPAD = 512


def _sc_gather_body(h_hbm, r_hbm, idx_t, idx_a, idx_r, outh, outa, outr,
                    idxbuf, rb0, rb1, rb2, rb3, rb4, rb5, rb6, rb7,
                    gsem, ssem):
    rowbufs = (rb0, rb1, rb2, rb3, rb4, rb5, rb6, rb7)
    wid = lax.axis_index("s") * NC + lax.axis_index("c")
    base = wid * PER_W

    def loop_body(j, carry):
        off = base + j * (CH * NBUF)
        pltpu.sync_copy(idx_t.at[pl.ds(off, CH * NBUF)], idxbuf)
        handles = [
            pltpu.async_copy(h_hbm.at[idxbuf.at[pl.ds(b * CH, CH)]],
                             rowbufs[b], gsem)
            for b in range(NBUF)
        ]
        for h in handles:
            h.wait()
        stores = [
            pltpu.async_copy(rowbufs[b], outh.at[pl.ds(off + b * CH, CH)],
                             ssem)
            for b in range(NBUF)
        ]
        for s in stores:
            s.wait()
        return carry

    lax.fori_loop(0, N_ITER, loop_body, 0)

    boff = wid * PER_W_B
    pltpu.sync_copy(idx_a.at[pl.ds(boff, PER_W_B)], idxbuf.at[pl.ds(0, PER_W_B)])
    pltpu.sync_copy(idx_r.at[pl.ds(boff, PER_W_B)], idxbuf.at[pl.ds(CH, PER_W_B)])
    ca = pltpu.async_copy(h_hbm.at[idxbuf.at[pl.ds(0, PER_W_B)]], rowbufs[0], gsem)
    cr = pltpu.async_copy(r_hbm.at[idxbuf.at[pl.ds(CH, PER_W_B)]], rowbufs[1], gsem)
    ca.wait()
    cr.wait()
    pltpu.sync_copy(rowbufs[0], outa.at[pl.ds(boff, PER_W_B)])
    pltpu.sync_copy(rowbufs[1], outr.at[pl.ds(boff, PER_W_B)])


@functools.lru_cache(maxsize=None)
def _build_sc_gather():
    return pl.kernel(
        _sc_gather_body,
        out_type=[
            jax.ShapeDtypeStruct((ROWS_H, DIM), jnp.float32),
            jax.ShapeDtypeStruct((B, DIM), jnp.float32),
            jax.ShapeDtypeStruct((B, DIM), jnp.float32),
        ],
        mesh=plsc.VectorSubcoreMesh(core_axis_name="c", subcore_axis_name="s",
                                    num_cores=NC, num_subcores=NS),
        scratch_types=(
            [pltpu.VMEM((CH * NBUF,), jnp.int32)]
            + [pltpu.VMEM((CH, DIM), jnp.float32) for _ in range(NBUF)]
            + [pltpu.SemaphoreType.DMA, pltpu.SemaphoreType.DMA]
        ),
        compiler_params=pltpu.CompilerParams(use_tc_tiling_on_sc=False),
    )


def _tc_body(ht_ref, a_ref, rq_ref, pre_ref, pro_ref, rhi_ref, rlo_ref,
             w1a_ref, bd1_ref, bd2_ref, b1_ref, b2_ref, out_ref):
    f32 = jnp.float32
    pre = pre_ref[0, 0, :]
    pro = pro_ref[0, 0, :]
    iota = lax.broadcasted_iota(jnp.int32, (NPAIR, RPAD), 1)
    oh = jnp.concatenate(
        [(iota == pre.reshape(NPAIR, 1)).astype(jnp.bfloat16),
         (iota == pro.reshape(NPAIR, 1)).astype(jnp.bfloat16)], axis=1)
    rt = jnp.dot(oh, rhi_ref[...], preferred_element_type=f32)
    rt = rt + jnp.dot(oh, rlo_ref[...], preferred_element_type=f32)

    rq = rq_ref[0]
    u0 = lax.dot_general(rq, w1a_ref[...], (((1,), (1,)), ((), ())),
                         preferred_element_type=f32,
                         precision=lax.Precision.HIGHEST)
    u0d = jnp.concatenate([u0, u0], axis=1)
    u0e = jnp.broadcast_to(u0d.reshape(BBLK, 1, 2 * DIM),
                           (BBLK, K // 2, 2 * DIM)).reshape(NPAIR, 2 * DIM)
    vk = jnp.dot(rt, bd1_ref[...], preferred_element_type=f32,
                 precision=lax.Precision.HIGHEST)
    act = jnp.maximum(u0e + vk + b1_ref[...], 0.0)
    m = jnp.dot(act, bd2_ref[...], preferred_element_type=f32,
                precision=lax.Precision.HIGHEST) + b2_ref[...]

    ht = ht_ref[0]
    p1 = jnp.sum((m * ht).reshape(BBLK, K // 2, 2 * DIM), axis=1)
    p2 = jnp.sum((m * rt).reshape(BBLK, K // 2, 2 * DIM), axis=1)
    s1 = p1[:, :DIM] + p1[:, DIM:]
    s2 = p2[:, :DIM] + p2[:, DIM:]
    a = a_ref[0]
    fre = s1 - a * s2
    query = a * rq
    refn = jnp.sum(jnp.abs(fre), axis=1, keepdims=True)
    qn = jnp.sum(jnp.abs(query), axis=1, keepdims=True)
    out_ref[0] = query + fre / (1e-9 + refn / qn * 2.5)


def _tc_call(ht, a, rq, pre, pro, rhi, rlo, w1a, bd1, bd2, b1, b2):
    return pl.pallas_call(
        _tc_body,
        grid=(GRID,),
        in_specs=[
            pl.BlockSpec((1, NPAIR, 2 * DIM), lambda i: (i, 0, 0)),
            pl.BlockSpec((1, BBLK, DIM), lambda i: (i, 0, 0)),
            pl.BlockSpec((1, BBLK, DIM), lambda i: (i, 0, 0)),
            pl.BlockSpec((1, 1, NPAIR), lambda i: (i, 0, 0)),
            pl.BlockSpec((1, 1, NPAIR), lambda i: (i, 0, 0)),
            pl.BlockSpec((2 * RPAD, 2 * DIM), lambda i: (0, 0)),
            pl.BlockSpec((2 * RPAD, 2 * DIM), lambda i: (0, 0)),
            pl.BlockSpec((DIM, DIM), lambda i: (0, 0)),
            pl.BlockSpec((2 * DIM, 2 * DIM), lambda i: (0, 0)),
            pl.BlockSpec((2 * DIM, 2 * DIM), lambda i: (0, 0)),
            pl.BlockSpec((1, 2 * DIM), lambda i: (0, 0)),
            pl.BlockSpec((1, 2 * DIM), lambda i: (0, 0)),
        ],
        out_specs=pl.BlockSpec((1, BBLK, DIM), lambda i: (i, 0, 0)),
        out_shape=jax.ShapeDtypeStruct((GRID, BBLK, DIM), jnp.float32),
    )(ht, a, rq, pre, pro, rhi, rlo, w1a, bd1, bd2, b1, b2)


def kernel(h_table, r_table, W1, b1, W2, b2, anchors, rel_0, p1_target, p1_rel):
    f32 = jnp.float32
    idx_t = p1_target.reshape(-1).astype(jnp.int32)
    idx_a = anchors.astype(jnp.int32)
    idx_r = rel_0.astype(jnp.int32)
    ht2, a_rows, rq_rows = _build_sc_gather()(h_table, r_table, idx_t, idx_a, idx_r)

    rhi = r_table.astype(jnp.bfloat16)
    rlo = (r_table - rhi.astype(f32)).astype(jnp.bfloat16)
    z = jnp.zeros((RPAD - (N_REL + 1), DIM), jnp.bfloat16)
    zc = jnp.zeros((RPAD, DIM), jnp.bfloat16)
    rhi_cat = jnp.concatenate(
        [jnp.concatenate([jnp.concatenate([rhi, z], 0), zc], 1),
         jnp.concatenate([zc, jnp.concatenate([rhi, z], 0)], 1)], axis=0)
    rlo_cat = jnp.concatenate(
        [jnp.concatenate([jnp.concatenate([rlo, z], 0), zc], 1),
         jnp.concatenate([zc, jnp.concatenate([rlo, z], 0)], 1)], axis=0)

    w1a = W1[:, :DIM]
    w1bt = W1[:, DIM:].T
    zw = jnp.zeros((DIM, DIM), f32)
    bd1 = jnp.concatenate(
        [jnp.concatenate([w1bt, zw], 1), jnp.concatenate([zw, w1bt], 1)], 0)
    w2t = W2.T
    bd2 = jnp.concatenate(
        [jnp.concatenate([w2t, zw], 1), jnp.concatenate([zw, w2t], 1)], 0)
    b1c = jnp.concatenate([b1, b1]).reshape(1, 2 * DIM)
    b2c = jnp.concatenate([b2, b2]).reshape(1, 2 * DIM)

    pre = p1_rel[:, 0::2].astype(jnp.int32).reshape(GRID, 1, NPAIR)
    pro = p1_rel[:, 1::2].astype(jnp.int32).reshape(GRID, 1, NPAIR)

    out = _tc_call(
        ht2.reshape(-1, 2 * DIM).reshape(GRID, NPAIR, 2 * DIM),
        a_rows.reshape(GRID, BBLK, DIM),
        rq_rows.reshape(GRID, BBLK, DIM),
        pre, pro, rhi_cat, rlo_cat, w1a, bd1, bd2, b1c, b2c,
    )
    return out.reshape(B, DIM)
